# baseline (speedup 1.0000x reference)
import functools

import jax
import jax.numpy as jnp
from jax import lax
from jax.experimental import pallas as pl
from jax.experimental.pallas import tpu as pltpu
from jax.experimental.pallas import tpu_sc as plsc

NC = 2
NS = 16
NW = NC * NS
CH = 128
DEG_W = 8


def _make_edge_agg(n_pad, d, kk, ch, nstage):
    rpt = n_pad // NS
    kp = kk // nstage
    assert kp * nstage == kk
    CR = 128
    assert ch >= CR and rpt % CR == 0

    mesh = plsc.VectorSubcoreMesh(core_axis_name="c", subcore_axis_name="s")

    @functools.partial(
        pl.kernel,
        out_type=jax.ShapeDtypeStruct((NC, n_pad, d), jnp.float32),
        mesh=mesh,
        compiler_params=pltpu.CompilerParams(use_tc_tiling_on_sc=False),
        scratch_types=[
            pltpu.VMEM((kp + 2, ch), jnp.int32),
            pltpu.VMEM((kp + 2, ch), jnp.int32),
            pltpu.VMEM((ch, d), jnp.float32),
            pltpu.VMEM_SHARED((n_pad, d), jnp.float32),
            pltpu.SemaphoreType.DMA,
        ],
    )
    def body(z_hbm, src_hbm, dst_hbm, zrows_hbm, out_hbm, src_v, dst_v,
             buf, acc, sem):
        c = lax.axis_index("c")
        s = lax.axis_index("s")
        wid = s * NC + c

        pltpu.sync_copy(zrows_hbm, buf)

        def zcp(i, carry):
            pltpu.sync_copy(buf.at[pl.ds(0, CR)],
                            acc.at[pl.ds(s * rpt + i * CR, CR)])
            return carry

        lax.fori_loop(0, rpt // CR, zcp, 0)

        plsc.subcore_barrier()

        for h in range(nstage):
            pltpu.sync_copy(src_hbm.at[wid, pl.ds(h * kp, kp + 2)], src_v)
            pltpu.sync_copy(dst_hbm.at[wid, pl.ds(h * kp, kp + 2)], dst_v)

            def step(j, carry):
                pltpu.async_copy(z_hbm.at[src_v.at[j]], buf, sem).wait()
                pltpu.sync_copy(buf, acc.at[dst_v.at[j]], add=True)
                return carry

            lax.fori_loop(0, kp, step, 0)

        plsc.subcore_barrier()

        def ocp(i, carry):
            pltpu.sync_copy(acc.at[pl.ds(s * rpt + i * CR, CR)],
                            buf.at[pl.ds(0, CR)])
            pltpu.sync_copy(buf.at[pl.ds(0, CR)],
                            out_hbm.at[c, pl.ds(s * rpt + i * CR, CR)])
            return carry

        lax.fori_loop(0, rpt // CR, ocp, 0)

    return body


def _make_deg(n_pad, k):
    rpt = n_pad // NS
    mesh = plsc.VectorSubcoreMesh(core_axis_name="c", subcore_axis_name="s")

    @functools.partial(
        pl.kernel,
        out_type=jax.ShapeDtypeStruct((NC, n_pad, DEG_W), jnp.float32),
        mesh=mesh,
        compiler_params=pltpu.CompilerParams(use_tc_tiling_on_sc=False),
        scratch_types=[
            pltpu.VMEM((k, CH), jnp.int32),
            pltpu.VMEM((CH, DEG_W), jnp.float32),
            pltpu.VMEM((CH, DEG_W), jnp.float32),
            pltpu.VMEM_SHARED((n_pad, DEG_W), jnp.float32),
        ],
    )
    def body(dst_hbm, zrows_hbm, onerows_hbm, out_hbm, dst_v, zbuf, onebuf, acc):
        c = lax.axis_index("c")
        s = lax.axis_index("s")
        wid = s * NC + c

        pltpu.sync_copy(zrows_hbm, zbuf)
        pltpu.sync_copy(onerows_hbm, onebuf)

        def zcp(i, carry):
            pltpu.sync_copy(zbuf, acc.at[pl.ds(s * rpt + i * CH, CH)])
            return carry

        lax.fori_loop(0, rpt // CH, zcp, 0)

        pltpu.sync_copy(dst_hbm.at[wid], dst_v)

        plsc.subcore_barrier()

        def step(j, carry):
            pltpu.sync_copy(onebuf, acc.at[dst_v.at[j]], add=True)
            return carry

        lax.fori_loop(0, k, step, 0)

        plsc.subcore_barrier()

        def ocp(i, carry):
            pltpu.sync_copy(acc.at[pl.ds(s * rpt + i * CH, CH)], zbuf)
            pltpu.sync_copy(zbuf, out_hbm.at[c, pl.ds(s * rpt + i * CH, CH)])
            return carry

        lax.fori_loop(0, rpt // CH, ocp, 0)

    return body


BLK = 1024


def _tc_matmul(xp, w):
    n_pad, din = xp.shape
    dout = w.shape[1]

    def body(xr, wr, outr):
        outr[...] = jnp.dot(xr[...], wr[...], preferred_element_type=jnp.float32)

    return pl.pallas_call(
        body,
        grid=(n_pad // BLK,),
        in_specs=[
            pl.BlockSpec((BLK, din), lambda i: (i, 0)),
            pl.BlockSpec((din, dout), lambda i: (0, 0)),
        ],
        out_specs=pl.BlockSpec((BLK, dout), lambda i: (i, 0)),
        out_shape=jax.ShapeDtypeStruct((n_pad, dout), jnp.float32),
    )(xp, w)


def _dis_block(gr):
    deg = gr[0, :, 0:1] + gr[1, :, 0:1] + 1.0
    return lax.rsqrt(deg)


def _tc_scale(xw, degacc):
    n_pad, d = xw.shape

    def body(xr, gr, outr):
        outr[...] = xr[...] * _dis_block(gr)

    return pl.pallas_call(
        body,
        grid=(n_pad // BLK,),
        in_specs=[
            pl.BlockSpec((BLK, d), lambda i: (i, 0)),
            pl.BlockSpec((NC, BLK, DEG_W), lambda i: (0, i, 0)),
        ],
        out_specs=pl.BlockSpec((BLK, d), lambda i: (i, 0)),
        out_shape=jax.ShapeDtypeStruct((n_pad, d), jnp.float32),
    )(xw, degacc)


def _tc_layer(z, agg, degacc, b, w, final_bias=None):
    n_pad, d = z.shape
    dout = w.shape[1]
    is_final = final_bias is not None

    def body(zr, ar, gr, br, wr, *rest):
        dis = _dis_block(gr)
        h = jnp.tanh((zr[...] + ar[0] + ar[1]) * dis + br[...])
        if is_final:
            b3r, outr = rest
            outr[...] = jnp.dot(h, wr[...], preferred_element_type=jnp.float32) + b3r[...]
        else:
            (outr,) = rest
            outr[...] = jnp.dot(h, wr[...], preferred_element_type=jnp.float32) * dis

    in_specs = [
        pl.BlockSpec((BLK, d), lambda i: (i, 0)),
        pl.BlockSpec((NC, BLK, d), lambda i: (0, i, 0)),
        pl.BlockSpec((NC, BLK, DEG_W), lambda i: (0, i, 0)),
        pl.BlockSpec((1, d), lambda i: (0, 0)),
        pl.BlockSpec((d, dout), lambda i: (0, 0)),
    ]
    args = [z, agg, degacc, b.reshape(1, d), w]
    if is_final:
        in_specs.append(pl.BlockSpec((1, dout), lambda i: (0, 0)))
        args.append(final_bias.reshape(1, dout))

    return pl.pallas_call(
        body,
        grid=(n_pad // BLK,),
        in_specs=in_specs,
        out_specs=pl.BlockSpec((BLK, dout), lambda i: (i, 0)),
        out_shape=jax.ShapeDtypeStruct((n_pad, dout), jnp.float32),
    )(*args)


def kernel(x, edge_index, W1, b1, W2, b2, W3, b3):
    n, d_in = x.shape
    e = edge_index.shape[1]

    n_pad = ((n + NS * CH - 1) // (NS * CH)) * (NS * CH)

    src = edge_index[0].astype(jnp.int32)
    dst = edge_index[1].astype(jnp.int32)
    safe_src = n_pad - 2
    safe_dst = n_pad - 1

    def edge_slabs(ch):
        k = (e + NW * ch - 1) // (NW * ch)
        kk = k + (k % 2)
        e_pad = NW * kk * ch
        pad_src = jnp.full((e_pad - e,), safe_src, jnp.int32)
        pad_dst = jnp.full((e_pad - e,), safe_dst, jnp.int32)
        src_p = jnp.concatenate([
            jnp.concatenate([src, pad_src]).reshape(NW, kk, ch),
            jnp.full((NW, 2, ch), safe_src, jnp.int32)], axis=1)
        dst_p = jnp.concatenate([
            jnp.concatenate([dst, pad_dst]).reshape(NW, kk, ch),
            jnp.full((NW, 2, ch), safe_dst, jnp.int32)], axis=1)
        return kk, src_p, dst_p

    kk1, src_p1, dst_p1 = edge_slabs(CH)
    kkA, src_pA, dst_pA = edge_slabs(512)
    kkB, src_pB, dst_pB = edge_slabs(256)

    x_pad = jnp.zeros((n_pad, d_in), jnp.float32).at[:n].set(x)

    zrows_deg = jnp.zeros((CH, DEG_W), jnp.float32)
    onerows = jnp.zeros((CH, DEG_W), jnp.float32).at[:, 0].set(1.0)

    degacc = _make_deg(n_pad, kk1 + 2)(dst_p1, zrows_deg, onerows)
    xw1 = _tc_matmul(x_pad, W1)
    z1 = _tc_scale(xw1, degacc)

    d1 = W1.shape[1]
    agg1 = _make_edge_agg(n_pad, d1, kkA, 512, 2)(
        z1, src_pA, dst_pA, jnp.zeros((512, d1), jnp.float32))
    z2 = _tc_layer(z1, agg1, degacc, b1, W2)

    d2 = W2.shape[1]
    agg2 = _make_edge_agg(n_pad, d2, kkB, 256, 2)(
        z2, src_pB, dst_pB, jnp.zeros((256, d2), jnp.float32))
    out_pad = _tc_layer(z2, agg2, degacc, b2, W3, final_bias=b3)

    return out_pad[:n]


# speedup vs baseline: 1.2866x; 1.2866x over previous
import functools

import jax
import jax.numpy as jnp
from jax import lax
from jax.experimental import pallas as pl
from jax.experimental.pallas import tpu as pltpu
from jax.experimental.pallas import tpu_sc as plsc

NC = 2
NS = 16
NW = NC * NS
CH = 128
DEG_W = 8


def _make_edge_agg(n_pad, d, kk, ch, nstage):
    rpt = n_pad // NS
    kp = kk // nstage
    assert kp * nstage == kk
    CR = 128
    assert ch >= CR and rpt % CR == 0

    mesh = plsc.VectorSubcoreMesh(core_axis_name="c", subcore_axis_name="s")

    @functools.partial(
        pl.kernel,
        out_type=jax.ShapeDtypeStruct((NC, n_pad, d), jnp.float32),
        mesh=mesh,
        compiler_params=pltpu.CompilerParams(use_tc_tiling_on_sc=False),
        scratch_types=[
            pltpu.VMEM((kp + 2, ch), jnp.int32),
            pltpu.VMEM((kp + 2, ch), jnp.int32),
            pltpu.VMEM((ch, d), jnp.float32),
            pltpu.VMEM_SHARED((n_pad, d), jnp.float32),
            pltpu.SemaphoreType.DMA,
        ],
    )
    def body(z_hbm, src_hbm, dst_hbm, zrows_hbm, out_hbm, src_v, dst_v,
             buf, acc, sem):
        c = lax.axis_index("c")
        s = lax.axis_index("s")
        wid = s * NC + c

        pltpu.sync_copy(zrows_hbm, buf)

        def zcp(i, carry):
            pltpu.sync_copy(buf.at[pl.ds(0, CR)],
                            acc.at[pl.ds(s * rpt + i * CR, CR)])
            return carry

        lax.fori_loop(0, rpt // CR, zcp, 0)

        plsc.subcore_barrier()

        for h in range(nstage):
            pltpu.sync_copy(src_hbm.at[wid, pl.ds(h * kp, kp + 2)], src_v)
            pltpu.sync_copy(dst_hbm.at[wid, pl.ds(h * kp, kp + 2)], dst_v)

            def step(j, carry):
                pltpu.async_copy(z_hbm.at[src_v.at[j]], buf, sem).wait()
                pltpu.sync_copy(buf, acc.at[dst_v.at[j]], add=True)
                return carry

            lax.fori_loop(0, kp, step, 0)

        plsc.subcore_barrier()

        def ocp(i, carry):
            pltpu.sync_copy(acc.at[pl.ds(s * rpt + i * CR, CR)],
                            buf.at[pl.ds(0, CR)])
            pltpu.sync_copy(buf.at[pl.ds(0, CR)],
                            out_hbm.at[c, pl.ds(s * rpt + i * CR, CR)])
            return carry

        lax.fori_loop(0, rpt // CR, ocp, 0)

    return body


def _make_deg(n_pad, k):
    rpt = n_pad // NS
    mesh = plsc.VectorSubcoreMesh(core_axis_name="c", subcore_axis_name="s")

    @functools.partial(
        pl.kernel,
        out_type=jax.ShapeDtypeStruct((NC, n_pad, DEG_W), jnp.float32),
        mesh=mesh,
        compiler_params=pltpu.CompilerParams(use_tc_tiling_on_sc=False),
        scratch_types=[
            pltpu.VMEM((k, CH), jnp.int32),
            pltpu.VMEM((CH, DEG_W), jnp.float32),
            pltpu.VMEM((CH, DEG_W), jnp.float32),
            pltpu.VMEM_SHARED((n_pad, DEG_W), jnp.float32),
        ],
    )
    def body(dst_hbm, zrows_hbm, onerows_hbm, out_hbm, dst_v, zbuf, onebuf, acc):
        c = lax.axis_index("c")
        s = lax.axis_index("s")
        wid = s * NC + c

        pltpu.sync_copy(zrows_hbm, zbuf)
        pltpu.sync_copy(onerows_hbm, onebuf)

        def zcp(i, carry):
            pltpu.sync_copy(zbuf, acc.at[pl.ds(s * rpt + i * CH, CH)])
            return carry

        lax.fori_loop(0, rpt // CH, zcp, 0)

        pltpu.sync_copy(dst_hbm.at[wid], dst_v)

        plsc.subcore_barrier()

        def step(j, carry):
            pltpu.sync_copy(onebuf, acc.at[dst_v.at[j]], add=True)
            return carry

        lax.fori_loop(0, k, step, 0)

        plsc.subcore_barrier()

        def ocp(i, carry):
            pltpu.sync_copy(acc.at[pl.ds(s * rpt + i * CH, CH)], zbuf)
            pltpu.sync_copy(zbuf, out_hbm.at[c, pl.ds(s * rpt + i * CH, CH)])
            return carry

        lax.fori_loop(0, rpt // CH, ocp, 0)

    return body


BLK = 1024


def _tc_matmul(xp, w):
    n_pad, din = xp.shape
    dout = w.shape[1]

    def body(xr, wr, outr):
        outr[...] = jnp.dot(xr[...], wr[...], preferred_element_type=jnp.float32)

    return pl.pallas_call(
        body,
        grid=(n_pad // BLK,),
        in_specs=[
            pl.BlockSpec((BLK, din), lambda i: (i, 0)),
            pl.BlockSpec((din, dout), lambda i: (0, 0)),
        ],
        out_specs=pl.BlockSpec((BLK, dout), lambda i: (i, 0)),
        out_shape=jax.ShapeDtypeStruct((n_pad, dout), jnp.float32),
    )(xp, w)


def _dis_block(gr):
    deg = gr[0, :, 0:1] + gr[1, :, 0:1] + 1.0
    return lax.rsqrt(deg)


def _tc_scale(xw, degacc):
    n_pad, d = xw.shape

    def body(xr, gr, outr):
        outr[...] = xr[...] * _dis_block(gr)

    return pl.pallas_call(
        body,
        grid=(n_pad // BLK,),
        in_specs=[
            pl.BlockSpec((BLK, d), lambda i: (i, 0)),
            pl.BlockSpec((NC, BLK, DEG_W), lambda i: (0, i, 0)),
        ],
        out_specs=pl.BlockSpec((BLK, d), lambda i: (i, 0)),
        out_shape=jax.ShapeDtypeStruct((n_pad, d), jnp.float32),
    )(xw, degacc)


def _tc_layer(z, agg, degacc, b, w, final_bias=None):
    n_pad, d = z.shape
    dout = w.shape[1]
    is_final = final_bias is not None

    def body(zr, ar, gr, br, wr, *rest):
        dis = _dis_block(gr)
        h = jnp.tanh((zr[...] + ar[0] + ar[1]) * dis + br[...])
        if is_final:
            b3r, outr = rest
            outr[...] = jnp.dot(h, wr[...], preferred_element_type=jnp.float32) + b3r[...]
        else:
            (outr,) = rest
            outr[...] = jnp.dot(h, wr[...], preferred_element_type=jnp.float32) * dis

    in_specs = [
        pl.BlockSpec((BLK, d), lambda i: (i, 0)),
        pl.BlockSpec((NC, BLK, d), lambda i: (0, i, 0)),
        pl.BlockSpec((NC, BLK, DEG_W), lambda i: (0, i, 0)),
        pl.BlockSpec((1, d), lambda i: (0, 0)),
        pl.BlockSpec((d, dout), lambda i: (0, 0)),
    ]
    args = [z, agg, degacc, b.reshape(1, d), w]
    if is_final:
        in_specs.append(pl.BlockSpec((1, dout), lambda i: (0, 0)))
        args.append(final_bias.reshape(1, dout))

    return pl.pallas_call(
        body,
        grid=(n_pad // BLK,),
        in_specs=in_specs,
        out_specs=pl.BlockSpec((BLK, dout), lambda i: (i, 0)),
        out_shape=jax.ShapeDtypeStruct((n_pad, dout), jnp.float32),
    )(*args)


def kernel(x, edge_index, W1, b1, W2, b2, W3, b3):
    n, d_in = x.shape
    e = edge_index.shape[1]

    n_pad = ((n + NS * CH - 1) // (NS * CH)) * (NS * CH)

    src = edge_index[0].astype(jnp.int32)
    dst = edge_index[1].astype(jnp.int32)
    safe_src = n_pad - 2
    safe_dst = n_pad - 1

    def edge_slabs(ch):
        k = (e + NW * ch - 1) // (NW * ch)
        kk = k + (k % 2)
        e_pad = NW * kk * ch
        pad_src = jnp.full((e_pad - e,), safe_src, jnp.int32)
        pad_dst = jnp.full((e_pad - e,), safe_dst, jnp.int32)
        src_p = jnp.concatenate([
            jnp.concatenate([src, pad_src]).reshape(NW, kk, ch),
            jnp.full((NW, 2, ch), safe_src, jnp.int32)], axis=1)
        dst_p = jnp.concatenate([
            jnp.concatenate([dst, pad_dst]).reshape(NW, kk, ch),
            jnp.full((NW, 2, ch), safe_dst, jnp.int32)], axis=1)
        return kk, src_p, dst_p

    kk1, src_p1, dst_p1 = edge_slabs(CH)

    x_pad = jnp.zeros((n_pad, d_in), jnp.float32).at[:n].set(x)

    zrows_deg = jnp.zeros((CH, DEG_W), jnp.float32)
    onerows = jnp.zeros((CH, DEG_W), jnp.float32).at[:, 0].set(1.0)

    degacc = _make_deg(n_pad, kk1 + 2)(dst_p1, zrows_deg, onerows)
    xw1 = _tc_matmul(x_pad, W1)
    z1 = _tc_scale(xw1, degacc)

    d1 = W1.shape[1]
    agg1 = _make_edge_agg(n_pad, d1, kk1, CH, 1)(
        z1, src_p1, dst_p1, jnp.zeros((CH, d1), jnp.float32))
    z2 = _tc_layer(z1, agg1, degacc, b1, W2)

    d2 = W2.shape[1]
    agg2 = _make_edge_agg(n_pad, d2, kk1, CH, 1)(
        z2, src_p1, dst_p1, jnp.zeros((CH, d2), jnp.float32))
    out_pad = _tc_layer(z2, agg2, degacc, b2, W3, final_bias=b3)

    return out_pad[:n]


# speedup vs baseline: 1.7165x; 1.3342x over previous
import functools

import jax
import jax.numpy as jnp
from jax import lax
from jax.experimental import pallas as pl
from jax.experimental.pallas import tpu as pltpu
from jax.experimental.pallas import tpu_sc as plsc

NC = 2
NS = 16
NW = NC * NS
CH = 128
DEG_W = 16


def _make_edge_agg(n_pad, d, k):
    rpt = n_pad // NS

    mesh = plsc.VectorSubcoreMesh(core_axis_name="c", subcore_axis_name="s")

    @functools.partial(
        pl.kernel,
        out_type=jax.ShapeDtypeStruct((NC, n_pad, d), jnp.float32),
        mesh=mesh,
        compiler_params=pltpu.CompilerParams(use_tc_tiling_on_sc=False),
        scratch_types=[
            pltpu.VMEM((k, CH), jnp.int32),
            pltpu.VMEM((k, CH), jnp.int32),
            pltpu.VMEM((CH, d), jnp.float32),
            pltpu.VMEM_SHARED((n_pad, d), jnp.float32),
            pltpu.SemaphoreType.DMA,
        ],
    )
    def body(z_hbm, src_hbm, dst_hbm, zrows_hbm, out_hbm, src_v, dst_v,
             buf, acc, sem):
        c = lax.axis_index("c")
        s = lax.axis_index("s")
        wid = s * NC + c

        pltpu.sync_copy(zrows_hbm, buf)

        def zcp(i, carry):
            pltpu.sync_copy(buf, acc.at[pl.ds(s * rpt + i * CH, CH)])
            return carry

        lax.fori_loop(0, rpt // CH, zcp, 0)

        pltpu.sync_copy(src_hbm.at[wid], src_v)
        pltpu.sync_copy(dst_hbm.at[wid], dst_v)

        plsc.subcore_barrier()

        def step(j, carry):
            pltpu.async_copy(z_hbm.at[src_v.at[j]], buf, sem).wait()
            pltpu.sync_copy(buf, acc.at[dst_v.at[j]], add=True)
            return carry

        lax.fori_loop(0, k, step, 0)

        plsc.subcore_barrier()

        def ocp(i, carry):
            pltpu.sync_copy(acc.at[pl.ds(s * rpt + i * CH, CH)], buf)
            pltpu.sync_copy(buf, out_hbm.at[c, pl.ds(s * rpt + i * CH, CH)])
            return carry

        lax.fori_loop(0, rpt // CH, ocp, 0)

    return body


def _make_deg(n_pad, k):
    rpt = n_pad // NS
    mesh = plsc.VectorSubcoreMesh(core_axis_name="c", subcore_axis_name="s")

    @functools.partial(
        pl.kernel,
        out_type=jax.ShapeDtypeStruct((NC, n_pad, DEG_W), jnp.float32),
        mesh=mesh,
        compiler_params=pltpu.CompilerParams(use_tc_tiling_on_sc=False),
        scratch_types=[
            pltpu.VMEM((k, CH), jnp.int32),
            pltpu.VMEM((CH, DEG_W), jnp.float32),
            pltpu.VMEM((CH, DEG_W), jnp.float32),
            pltpu.VMEM_SHARED((n_pad, DEG_W), jnp.float32),
        ],
    )
    def body(dst_hbm, zrows_hbm, onerows_hbm, out_hbm, dst_v, zbuf, onebuf, acc):
        c = lax.axis_index("c")
        s = lax.axis_index("s")
        wid = s * NC + c

        pltpu.sync_copy(zrows_hbm, zbuf)
        pltpu.sync_copy(onerows_hbm, onebuf)

        def zcp(i, carry):
            pltpu.sync_copy(zbuf, acc.at[pl.ds(s * rpt + i * CH, CH)])
            return carry

        lax.fori_loop(0, rpt // CH, zcp, 0)

        pltpu.sync_copy(dst_hbm.at[wid], dst_v)

        plsc.subcore_barrier()

        def step(j, carry):
            pltpu.sync_copy(onebuf, acc.at[dst_v.at[j]], add=True)
            return carry

        lax.fori_loop(0, k, step, 0)

        plsc.subcore_barrier()

        def ocp(i, carry):
            pltpu.sync_copy(acc.at[pl.ds(s * rpt + i * CH, CH)], zbuf)
            pltpu.sync_copy(zbuf, out_hbm.at[c, pl.ds(s * rpt + i * CH, CH)])
            return carry

        lax.fori_loop(0, rpt // CH, ocp, 0)

    return body


BLK = 1024


def _tc_matmul(xp, w):
    n_pad, din = xp.shape
    dout = w.shape[1]

    def body(xr, wr, outr):
        outr[...] = jnp.dot(xr[...], wr[...], preferred_element_type=jnp.float32)

    return pl.pallas_call(
        body,
        grid=(n_pad // BLK,),
        in_specs=[
            pl.BlockSpec((BLK, din), lambda i: (i, 0)),
            pl.BlockSpec((din, dout), lambda i: (0, 0)),
        ],
        out_specs=pl.BlockSpec((BLK, dout), lambda i: (i, 0)),
        out_shape=jax.ShapeDtypeStruct((n_pad, dout), jnp.float32),
    )(xp, w)


def _dis_block(gr):
    deg = gr[0, :, 0:1] + gr[1, :, 0:1] + 1.0
    return lax.rsqrt(deg)


def _tc_scale(xw, degacc):
    n_pad, d = xw.shape

    def body(xr, gr, outr):
        outr[...] = xr[...] * _dis_block(gr)

    return pl.pallas_call(
        body,
        grid=(n_pad // BLK,),
        in_specs=[
            pl.BlockSpec((BLK, d), lambda i: (i, 0)),
            pl.BlockSpec((NC, BLK, DEG_W), lambda i: (0, i, 0)),
        ],
        out_specs=pl.BlockSpec((BLK, d), lambda i: (i, 0)),
        out_shape=jax.ShapeDtypeStruct((n_pad, d), jnp.float32),
    )(xw, degacc)


def _tc_layer(z, agg, degacc, b, w, final_bias=None):
    n_pad, d = z.shape
    dout = w.shape[1]
    is_final = final_bias is not None

    def body(zr, ar, gr, br, wr, *rest):
        dis = _dis_block(gr)
        h = jnp.tanh((zr[...] + ar[0] + ar[1]) * dis + br[...])
        if is_final:
            b3r, outr = rest
            outr[...] = jnp.dot(h, wr[...], preferred_element_type=jnp.float32) + b3r[...]
        else:
            (outr,) = rest
            outr[...] = jnp.dot(h, wr[...], preferred_element_type=jnp.float32) * dis

    in_specs = [
        pl.BlockSpec((BLK, d), lambda i: (i, 0)),
        pl.BlockSpec((NC, BLK, d), lambda i: (0, i, 0)),
        pl.BlockSpec((NC, BLK, DEG_W), lambda i: (0, i, 0)),
        pl.BlockSpec((1, d), lambda i: (0, 0)),
        pl.BlockSpec((d, dout), lambda i: (0, 0)),
    ]
    args = [z, agg, degacc, b.reshape(1, d), w]
    if is_final:
        in_specs.append(pl.BlockSpec((1, dout), lambda i: (0, 0)))
        args.append(final_bias.reshape(1, dout))

    return pl.pallas_call(
        body,
        grid=(n_pad // BLK,),
        in_specs=in_specs,
        out_specs=pl.BlockSpec((BLK, dout), lambda i: (i, 0)),
        out_shape=jax.ShapeDtypeStruct((n_pad, dout), jnp.float32),
    )(*args)


def kernel(x, edge_index, W1, b1, W2, b2, W3, b3):
    n, d_in = x.shape
    e = edge_index.shape[1]

    n_pad = ((n + NS * CH - 1) // (NS * CH)) * (NS * CH)

    src = edge_index[0].astype(jnp.int32)
    dst = edge_index[1].astype(jnp.int32)
    safe_src = n_pad - 2
    safe_dst = n_pad - 1

    k = (e + NW * CH - 1) // (NW * CH)
    e_pad = NW * k * CH
    pad_src = jnp.full((e_pad - e,), safe_src, jnp.int32)
    pad_dst = jnp.full((e_pad - e,), safe_dst, jnp.int32)
    src_p = jnp.concatenate([src, pad_src]).reshape(NW, k, CH)
    dst_p = jnp.concatenate([dst, pad_dst]).reshape(NW, k, CH)

    x_pad = jnp.zeros((n_pad, d_in), jnp.float32).at[:n].set(x)

    zrows_deg = jnp.zeros((CH, DEG_W), jnp.float32)
    onerows = jnp.zeros((CH, DEG_W), jnp.float32).at[:, 0].set(1.0)

    degacc = _make_deg(n_pad, k)(dst_p, zrows_deg, onerows)
    xw1 = _tc_matmul(x_pad, W1)
    z1 = _tc_scale(xw1, degacc)

    d1 = W1.shape[1]
    agg1 = _make_edge_agg(n_pad, d1, k)(
        z1, src_p, dst_p, jnp.zeros((CH, d1), jnp.float32))
    z2 = _tc_layer(z1, agg1, degacc, b1, W2)

    d2 = W2.shape[1]
    agg2 = _make_edge_agg(n_pad, d2, k)(
        z2, src_p, dst_p, jnp.zeros((CH, d2), jnp.float32))
    out_pad = _tc_layer(z2, agg2, degacc, b2, W3, final_bias=b3)

    return out_pad[:n]
